# speedup vs baseline: 2.2941x; 2.2941x over previous
import jax
import jax.numpy as jnp
from jax.experimental import pallas as pl

_BLK = 512


def _bcast_kernel(emb_ref, out_ref):
    out_ref[...] = jnp.broadcast_to(emb_ref[...][None, :, :], out_ref.shape)


def kernel(x, emb_weight):
    B, L, D = x.shape
    grid = (L // _BLK,)
    out = pl.pallas_call(
        _bcast_kernel,
        grid=grid,
        in_specs=[pl.BlockSpec((_BLK, D), lambda i: (i, 0))],
        out_specs=pl.BlockSpec((B, _BLK, D), lambda i: (0, i, 0)),
        out_shape=jax.ShapeDtypeStruct((B, L, D), emb_weight.dtype),
    )(emb_weight)
    return out


# speedup vs baseline: 2.3668x; 1.0317x over previous
import jax
import jax.numpy as jnp
from jax.experimental import pallas as pl

_BLK = 1024


def _bcast_kernel(emb_ref, out_ref):
    out_ref[...] = jnp.broadcast_to(emb_ref[...][None, :, :], out_ref.shape)


def kernel(x, emb_weight):
    B, L, D = x.shape
    grid = (L // _BLK,)
    out = pl.pallas_call(
        _bcast_kernel,
        grid=grid,
        in_specs=[pl.BlockSpec((_BLK, D), lambda i: (i, 0))],
        out_specs=pl.BlockSpec((B, _BLK, D), lambda i: (0, i, 0)),
        out_shape=jax.ShapeDtypeStruct((B, L, D), emb_weight.dtype),
    )(emb_weight)
    return out
